# baseline (speedup 1.0000x reference)
import functools

import jax
import jax.numpy as jnp
from jax import lax
from jax.experimental import pallas as pl
from jax.experimental.pallas import tpu as pltpu
from jax.experimental.pallas import tpu_sc as plsc

N = 50000
E = 160000
BATCH = 256
NB = 640
EPAD = NB * BATCH
NTILES = 16
BT = NB // NTILES
FP = 32
TFW = 64
NPAD = 50048
ACC_ROWS = 50176
DUMMY = N
ZROWS = 56
ZCOPIES = 56
DRAIN = 3128
ISTAGE = 8
ISTAGE2 = 8
ZTILE = ACC_ROWS // NTILES

BM = 2000
GRID = N // BM

_mesh = plsc.VectorSubcoreMesh(core_axis_name="c", subcore_axis_name="s")


@functools.partial(
    pl.kernel,
    mesh=_mesh,
    out_type=jax.ShapeDtypeStruct((4, NPAD, 16), jnp.float32),
    scratch_types=[
        pltpu.VMEM_SHARED((ACC_ROWS, 16), jnp.float32),
        pltpu.VMEM((ZROWS, 16), jnp.float32),
        pltpu.VMEM((ISTAGE, BATCH), jnp.int32),
        pltpu.VMEM((BATCH, 16), jnp.float32),
    ],
    compiler_params=pltpu.CompilerParams(use_tc_tiling_on_sc=False),
)
def _sc_counts(dst_all, out, acc, zbuf, didx, ones):
    c = lax.axis_index("c")
    s = lax.axis_index("s")
    zero16 = jnp.zeros((16,), jnp.float32)
    one16 = jnp.ones((16,), jnp.float32)

    def fill_z(i, carry):
        zbuf[i, :] = zero16
        return carry
    lax.fori_loop(0, ZROWS, fill_z, 0)

    def fill_o(i, carry):
        ones[i, :] = one16
        return carry
    lax.fori_loop(0, BATCH, fill_o, 0)

    def one_rel(i, carry):
        r = 2 * c + i

        def zcp(k, c2):
            pltpu.sync_copy(zbuf, acc.at[pl.ds(s * (ZROWS * ZCOPIES) + k * ZROWS, ZROWS)])
            return c2
        lax.fori_loop(0, ZCOPIES, zcp, 0)
        plsc.subcore_barrier()

        def chunk(q, c2):
            pltpu.sync_copy(dst_all.at[r].at[pl.ds(s * BT + q * ISTAGE, ISTAGE)], didx)

            def bat(j, c3):
                pltpu.sync_copy(ones, acc.at[didx.at[j]], add=True)
                return c3
            lax.fori_loop(0, ISTAGE, bat, 0)
            return c2
        lax.fori_loop(0, BT // ISTAGE, chunk, 0)
        plsc.subcore_barrier()
        pltpu.sync_copy(acc.at[pl.ds(s * DRAIN, DRAIN)],
                        out.at[r].at[pl.ds(s * DRAIN, DRAIN)])
        plsc.subcore_barrier()
        return carry
    lax.fori_loop(0, 2, one_rel, 0)


@functools.partial(
    pl.kernel,
    mesh=_mesh,
    out_type=[jax.ShapeDtypeStruct((2, NPAD, TFW), jnp.bfloat16),
              jax.ShapeDtypeStruct((2, NPAD, TFW), jnp.bfloat16)],
    scratch_types=[
        pltpu.VMEM_SHARED((ACC_ROWS, TFW), jnp.bfloat16),
        pltpu.VMEM((ISTAGE2, BATCH), jnp.int32),
        pltpu.VMEM((ISTAGE2, BATCH), jnp.int32),
        pltpu.VMEM((BATCH, TFW), jnp.bfloat16),
        pltpu.VMEM((BATCH, TFW), jnp.bfloat16),
        pltpu.VMEM((BATCH, TFW), jnp.bfloat16),
        pltpu.SemaphoreType.DMA,
        pltpu.SemaphoreType.DMA,
        pltpu.SemaphoreType.DMA,
        pltpu.SemaphoreType.DMA,
        pltpu.SemaphoreType.DMA,
        pltpu.SemaphoreType.DMA,
    ],
    compiler_params=pltpu.CompilerParams(use_tc_tiling_on_sc=False),
)
def _sc_scatter(tblA, srcA, dstA, tblB, srcB, dstB, zeros, outA, outB,
                acc, sidx, didx, rows0, rows1, rows2,
                gsem0, gsem1, gsem2, ssem0, ssem1, ssem2):
    c = lax.axis_index("c")
    s = lax.axis_index("s")
    p = c
    bufs = (rows0, rows1, rows2)
    gsems = (gsem0, gsem1, gsem2)
    ssems = (ssem0, ssem1, ssem2)

    for tbl, src2d, dst2d, out in ((tblA, srcA, dstA, outA),
                                   (tblB, srcB, dstB, outB)):
        pltpu.sync_copy(zeros.at[pl.ds(s * ZTILE, ZTILE)],
                        acc.at[pl.ds(s * ZTILE, ZTILE)])
        plsc.subcore_barrier()

        def chunk(q, c2, src2d=src2d, dst2d=dst2d, tbl=tbl):
            pltpu.sync_copy(src2d.at[pl.ds(s * BT + q * ISTAGE2, ISTAGE2)], sidx)
            pltpu.sync_copy(dst2d.at[pl.ds(s * BT + q * ISTAGE2, ISTAGE2)], didx)
            gp = [None] * 3
            sp = [None] * 3
            for j in range(2):
                gp[j] = pltpu.async_copy(tbl.at[p].at[sidx.at[j]], bufs[j], gsems[j])
            for j in range(ISTAGE2):
                b = j % 3
                gp[b].wait()
                sp[b] = pltpu.async_copy(bufs[b], acc.at[didx.at[j]], ssems[b],
                                         add=True)
                if j + 2 < ISTAGE2:
                    bl = (j + 2) % 3
                    if sp[bl] is not None:
                        sp[bl].wait()
                        sp[bl] = None
                    gp[bl] = pltpu.async_copy(tbl.at[p].at[sidx.at[j + 2]],
                                              bufs[bl], gsems[bl])
            for b in range(3):
                if sp[b] is not None:
                    sp[b].wait()
            return c2
        lax.fori_loop(0, BT // ISTAGE2, chunk, 0)
        plsc.subcore_barrier()
        pltpu.sync_copy(acc.at[pl.ds(s * DRAIN, DRAIN)],
                        out.at[p].at[pl.ds(s * DRAIN, DRAIN)])
        plsc.subcore_barrier()


def _mm_body(cnt_ref, x_ref, wg_ref, wl_ref, wr_ref, tg_ref, tl_ref, m_ref):
    x = x_ref[...]
    dinv = lax.rsqrt(cnt_ref[:, :1] + 1.0)
    hg = jnp.dot(x, wg_ref[...], preferred_element_type=jnp.float32) * dinv
    hl = jnp.dot(x, wl_ref[...], preferred_element_type=jnp.float32)
    m_ref[...] = jnp.dot(x, wr_ref[...],
                         preferred_element_type=jnp.float32).astype(jnp.bfloat16)
    for p in range(2):
        tg_ref[p, :, :] = hg[:, TFW * p:TFW * (p + 1)].astype(jnp.bfloat16)
        tl_ref[p, :, :] = hl[:, TFW * p:TFW * (p + 1)].astype(jnp.bfloat16)


def _mm(cnt, x, w_gcn, w_sage_l, w_sage_r):
    return pl.pallas_call(
        _mm_body,
        grid=(GRID,),
        in_specs=[
            pl.BlockSpec((BM, 16), lambda i: (i, 0)),
            pl.BlockSpec((BM, 128), lambda i: (i, 0)),
            pl.BlockSpec((128, 128), lambda i: (0, 0)),
            pl.BlockSpec((128, 128), lambda i: (0, 0)),
            pl.BlockSpec((128, 128), lambda i: (0, 0)),
        ],
        out_specs=[
            pl.BlockSpec((2, BM, TFW), lambda i: (0, i, 0)),
            pl.BlockSpec((2, BM, TFW), lambda i: (0, i, 0)),
            pl.BlockSpec((BM, 128), lambda i: (i, 0)),
        ],
        out_shape=[
            jax.ShapeDtypeStruct((2, N, TFW), jnp.bfloat16),
            jax.ShapeDtypeStruct((2, N, TFW), jnp.bfloat16),
            jax.ShapeDtypeStruct((N, 128), jnp.bfloat16),
        ],
    )(cnt, x, w_gcn, w_sage_l, w_sage_r)


def _relu_block(degc, cntc, sg, tg, ss, m, b):
    dinv = lax.rsqrt(degc[:, :1] + 1.0)
    icnt = 1.0 / jnp.maximum(cntc[:, :1], 1.0)
    cols = []
    for p in range(2):
        sgp = sg[p, :, :].astype(jnp.float32)
        tgp = tg[p, :, :].astype(jnp.float32)
        ssp = ss[p, :, :].astype(jnp.float32)
        mp = m[:, TFW * p:TFW * (p + 1)].astype(jnp.float32)
        v = (sgp + tgp) * dinv + ssp * icnt + mp + b[:, TFW * p:TFW * (p + 1)]
        cols.append(jnp.maximum(v, 0.0))
    return cols


def _combine_body(degc, cntc, sg, tg, ss, m, b, o):
    cols = _relu_block(degc, cntc, sg, tg, ss, m, b)
    for p in range(2):
        o[:, TFW * p:TFW * (p + 1)] = cols[p]


def _combine(degc, cntc, s_gcn, t_gcn, s_sage, m, bias):
    return pl.pallas_call(
        _combine_body,
        grid=(GRID,),
        in_specs=[
            pl.BlockSpec((BM, 16), lambda i: (i, 0)),
            pl.BlockSpec((BM, 16), lambda i: (i, 0)),
            pl.BlockSpec((2, BM, TFW), lambda i: (0, i, 0)),
            pl.BlockSpec((2, BM, TFW), lambda i: (0, i, 0)),
            pl.BlockSpec((2, BM, TFW), lambda i: (0, i, 0)),
            pl.BlockSpec((BM, 128), lambda i: (i, 0)),
            pl.BlockSpec((1, 128), lambda i: (0, 0)),
        ],
        out_specs=pl.BlockSpec((BM, 128), lambda i: (i, 0)),
        out_shape=jax.ShapeDtypeStruct((N, 128), jnp.float32),
    )(degc, cntc, s_gcn, t_gcn, s_sage, m, bias)


def _combine_final_body(degc, cntc, sg, tg, ss, m, b, w, bo, o):
    cols = _relu_block(degc, cntc, sg, tg, ss, m, b)
    x2 = jnp.concatenate(cols, axis=1)
    o[...] = jnp.dot(x2, w[...], preferred_element_type=jnp.float32) + bo[...]


def _combine_final(degc, cntc, s_gcn, t_gcn, s_sage, m, bias, w, bout):
    return pl.pallas_call(
        _combine_final_body,
        grid=(GRID,),
        in_specs=[
            pl.BlockSpec((BM, 16), lambda i: (i, 0)),
            pl.BlockSpec((BM, 16), lambda i: (i, 0)),
            pl.BlockSpec((2, BM, TFW), lambda i: (0, i, 0)),
            pl.BlockSpec((2, BM, TFW), lambda i: (0, i, 0)),
            pl.BlockSpec((2, BM, TFW), lambda i: (0, i, 0)),
            pl.BlockSpec((BM, 128), lambda i: (i, 0)),
            pl.BlockSpec((1, 128), lambda i: (0, 0)),
            pl.BlockSpec((128, 64), lambda i: (0, 0)),
            pl.BlockSpec((1, 64), lambda i: (0, 0)),
        ],
        out_specs=pl.BlockSpec((BM, 64), lambda i: (i, 0)),
        out_shape=jax.ShapeDtypeStruct((N, 64), jnp.float32),
    )(degc, cntc, s_gcn, t_gcn, s_sage, m, bias, w, bout)


def _final_body(x_ref, w_ref, b_ref, o_ref):
    o_ref[...] = (jnp.dot(x_ref[...], w_ref[...],
                          preferred_element_type=jnp.float32) + b_ref[...])


def _final(x, w, b):
    return pl.pallas_call(
        _final_body,
        grid=(GRID,),
        in_specs=[
            pl.BlockSpec((BM, 128), lambda i: (i, 0)),
            pl.BlockSpec((128, 64), lambda i: (0, 0)),
            pl.BlockSpec((1, 64), lambda i: (0, 0)),
        ],
        out_specs=pl.BlockSpec((BM, 64), lambda i: (i, 0)),
        out_shape=jax.ShapeDtypeStruct((N, 64), jnp.float32),
    )(x, w, b)


def _pad_edges(ei):
    pad = EPAD - E
    src = jnp.concatenate([ei[0].astype(jnp.int32),
                           jnp.zeros((pad,), jnp.int32)])
    dst = jnp.concatenate([ei[1].astype(jnp.int32),
                           jnp.full((pad,), DUMMY, jnp.int32)])
    return src.reshape(NB, BATCH), dst.reshape(NB, BATCH)


def kernel(x_general, x_rainfall, edge_index_gg, edge_index_gr,
           edge_index_rg, edge_index_rr, params):
    s_gg, d_gg = _pad_edges(edge_index_gg)
    s_gr, d_gr = _pad_edges(edge_index_gr)
    s_rg, d_rg = _pad_edges(edge_index_rg)
    s_rr, d_rr = _pad_edges(edge_index_rr)

    counts = _sc_counts(jnp.stack([d_gg, d_rg, d_rr, d_gr]))
    deg_g, cnt_rg = counts[0], counts[1]
    deg_r, cnt_gr = counts[2], counts[3]
    zeros = jnp.zeros((ACC_ROWS, TFW), jnp.bfloat16)

    wg, bg = params['lin_general']
    wr, br = params['lin_rainfall']

    xg, xr = x_general, x_rainfall
    layers = params['layers']
    for li, lp in enumerate(layers):
        tg_g, tl_g, m_g = _mm(deg_g, xg, lp['W_gg'], lp['Wl_gr'], lp['Wr_rg'])
        tg_r, tl_r, m_r = _mm(deg_r, xr, lp['W_rr'], lp['Wl_rg'], lp['Wr_gr'])
        s_gcn_g, s_sage_g = _sc_scatter(tg_g, s_gg, d_gg, tl_r, s_rg, d_rg, zeros)
        s_gcn_r, s_sage_r = _sc_scatter(tg_r, s_rr, d_rr, tl_g, s_gr, d_gr, zeros)
        bias_g = (lp['b_gg'] + lp['bl_rg']).reshape(1, 128)
        bias_r = (lp['b_rr'] + lp['bl_gr']).reshape(1, 128)
        if li + 1 < len(layers):
            xg = _combine(deg_g, cnt_rg, s_gcn_g, tg_g, s_sage_g, m_g, bias_g)
            xr = _combine(deg_r, cnt_gr, s_gcn_r, tg_r, s_sage_r, m_r, bias_r)
        else:
            gen_out = _combine_final(deg_g, cnt_rg, s_gcn_g, tg_g, s_sage_g,
                                     m_g, bias_g, wg, bg.reshape(1, 64))
            rain_out = _combine_final(deg_r, cnt_gr, s_gcn_r, tg_r, s_sage_r,
                                      m_r, bias_r, wr, br.reshape(1, 64))
    return (gen_out, rain_out)


# speedup vs baseline: 1.0460x; 1.0460x over previous
import functools

import jax
import jax.numpy as jnp
from jax import lax
from jax.experimental import pallas as pl
from jax.experimental.pallas import tpu as pltpu
from jax.experimental.pallas import tpu_sc as plsc

N = 50000
E = 160000
BATCH = 256
NB = 640
EPAD = NB * BATCH
NTILES = 16
BT = NB // NTILES
FP = 32
TFW = 64
NPAD = 50048
ACC_ROWS = 50176
DUMMY = N
ZROWS = 56
ZCOPIES = 56
DRAIN = 3128
ISTAGE = 8
ISTAGE2 = 8
ZTILE = ACC_ROWS // NTILES

BM = 2000
GRID = N // BM

_mesh = plsc.VectorSubcoreMesh(core_axis_name="c", subcore_axis_name="s")


@functools.partial(
    pl.kernel,
    mesh=_mesh,
    out_type=jax.ShapeDtypeStruct((4, NPAD, 16), jnp.float32),
    scratch_types=[
        pltpu.VMEM_SHARED((ACC_ROWS, 16), jnp.float32),
        pltpu.VMEM((ZROWS, 16), jnp.float32),
        pltpu.VMEM((ISTAGE, BATCH), jnp.int32),
        pltpu.VMEM((BATCH, 16), jnp.float32),
    ],
    compiler_params=pltpu.CompilerParams(use_tc_tiling_on_sc=False),
)
def _sc_counts(dst_all, out, acc, zbuf, didx, ones):
    c = lax.axis_index("c")
    s = lax.axis_index("s")
    zero16 = jnp.zeros((16,), jnp.float32)
    one16 = jnp.ones((16,), jnp.float32)

    def fill_z(i, carry):
        zbuf[i, :] = zero16
        return carry
    lax.fori_loop(0, ZROWS, fill_z, 0)

    def fill_o(i, carry):
        ones[i, :] = one16
        return carry
    lax.fori_loop(0, BATCH, fill_o, 0)

    def one_rel(i, carry):
        r = 2 * c + i

        def zcp(k, c2):
            pltpu.sync_copy(zbuf, acc.at[pl.ds(s * (ZROWS * ZCOPIES) + k * ZROWS, ZROWS)])
            return c2
        lax.fori_loop(0, ZCOPIES, zcp, 0)
        plsc.subcore_barrier()

        def chunk(q, c2):
            pltpu.sync_copy(dst_all.at[r].at[pl.ds(s * BT + q * ISTAGE, ISTAGE)], didx)

            def bat(j, c3):
                pltpu.sync_copy(ones, acc.at[didx.at[j]], add=True)
                return c3
            lax.fori_loop(0, ISTAGE, bat, 0)
            return c2
        lax.fori_loop(0, BT // ISTAGE, chunk, 0)
        plsc.subcore_barrier()
        pltpu.sync_copy(acc.at[pl.ds(s * DRAIN, DRAIN)],
                        out.at[r].at[pl.ds(s * DRAIN, DRAIN)])
        plsc.subcore_barrier()
        return carry
    lax.fori_loop(0, 2, one_rel, 0)


@functools.partial(
    pl.kernel,
    mesh=_mesh,
    out_type=jax.ShapeDtypeStruct((2, NPAD, TFW), jnp.bfloat16),
    scratch_types=[
        pltpu.VMEM_SHARED((ACC_ROWS, TFW), jnp.bfloat16),
        pltpu.VMEM((ISTAGE2, BATCH), jnp.int32),
        pltpu.VMEM((ISTAGE2, BATCH), jnp.int32),
        pltpu.VMEM((BATCH, TFW), jnp.bfloat16),
        pltpu.VMEM((BATCH, TFW), jnp.bfloat16),
        pltpu.VMEM((BATCH, TFW), jnp.bfloat16),
        pltpu.SemaphoreType.DMA,
        pltpu.SemaphoreType.DMA,
        pltpu.SemaphoreType.DMA,
        pltpu.SemaphoreType.DMA,
        pltpu.SemaphoreType.DMA,
        pltpu.SemaphoreType.DMA,
    ],
    compiler_params=pltpu.CompilerParams(use_tc_tiling_on_sc=False),
)
def _sc_scatter(tbl, src2d, dst2d, zeros, out,
                acc, sidx, didx, rows0, rows1, rows2,
                gsem0, gsem1, gsem2, ssem0, ssem1, ssem2):
    c = lax.axis_index("c")
    s = lax.axis_index("s")
    p = c
    bufs = (rows0, rows1, rows2)
    gsems = (gsem0, gsem1, gsem2)
    ssems = (ssem0, ssem1, ssem2)

    if True:
        pltpu.sync_copy(zeros.at[pl.ds(s * ZTILE, ZTILE)],
                        acc.at[pl.ds(s * ZTILE, ZTILE)])
        plsc.subcore_barrier()

        def chunk(q, c2):
            pltpu.sync_copy(src2d.at[pl.ds(s * BT + q * ISTAGE2, ISTAGE2)], sidx)
            pltpu.sync_copy(dst2d.at[pl.ds(s * BT + q * ISTAGE2, ISTAGE2)], didx)
            gp = [None] * 3
            sp = [None] * 3
            for j in range(2):
                gp[j] = pltpu.async_copy(tbl.at[p].at[sidx.at[j]], bufs[j], gsems[j])
            for j in range(ISTAGE2):
                b = j % 3
                gp[b].wait()
                sp[b] = pltpu.async_copy(bufs[b], acc.at[didx.at[j]], ssems[b],
                                         add=True)
                if j + 2 < ISTAGE2:
                    bl = (j + 2) % 3
                    if sp[bl] is not None:
                        sp[bl].wait()
                        sp[bl] = None
                    gp[bl] = pltpu.async_copy(tbl.at[p].at[sidx.at[j + 2]],
                                              bufs[bl], gsems[bl])
            for b in range(3):
                if sp[b] is not None:
                    sp[b].wait()
            return c2
        lax.fori_loop(0, BT // ISTAGE2, chunk, 0)
        plsc.subcore_barrier()
        pltpu.sync_copy(acc.at[pl.ds(s * DRAIN, DRAIN)],
                        out.at[p].at[pl.ds(s * DRAIN, DRAIN)])
        plsc.subcore_barrier()


def _mm_body(cnt_ref, x_ref, wg_ref, wl_ref, wr_ref, tg_ref, tl_ref, m_ref):
    x = x_ref[...]
    dinv = lax.rsqrt(cnt_ref[:, :1] + 1.0)
    hg = jnp.dot(x, wg_ref[...], preferred_element_type=jnp.float32) * dinv
    hl = jnp.dot(x, wl_ref[...], preferred_element_type=jnp.float32)
    m_ref[...] = jnp.dot(x, wr_ref[...],
                         preferred_element_type=jnp.float32).astype(jnp.bfloat16)
    for p in range(2):
        tg_ref[p, :, :] = hg[:, TFW * p:TFW * (p + 1)].astype(jnp.bfloat16)
        tl_ref[p, :, :] = hl[:, TFW * p:TFW * (p + 1)].astype(jnp.bfloat16)


def _mm(cnt, x, w_gcn, w_sage_l, w_sage_r):
    return pl.pallas_call(
        _mm_body,
        grid=(GRID,),
        in_specs=[
            pl.BlockSpec((BM, 16), lambda i: (i, 0)),
            pl.BlockSpec((BM, 128), lambda i: (i, 0)),
            pl.BlockSpec((128, 128), lambda i: (0, 0)),
            pl.BlockSpec((128, 128), lambda i: (0, 0)),
            pl.BlockSpec((128, 128), lambda i: (0, 0)),
        ],
        out_specs=[
            pl.BlockSpec((2, BM, TFW), lambda i: (0, i, 0)),
            pl.BlockSpec((2, BM, TFW), lambda i: (0, i, 0)),
            pl.BlockSpec((BM, 128), lambda i: (i, 0)),
        ],
        out_shape=[
            jax.ShapeDtypeStruct((2, N, TFW), jnp.bfloat16),
            jax.ShapeDtypeStruct((2, N, TFW), jnp.bfloat16),
            jax.ShapeDtypeStruct((N, 128), jnp.bfloat16),
        ],
    )(cnt, x, w_gcn, w_sage_l, w_sage_r)


def _relu_block(degc, cntc, sg, tg, ss, m, b):
    dinv = lax.rsqrt(degc[:, :1] + 1.0)
    icnt = 1.0 / jnp.maximum(cntc[:, :1], 1.0)
    cols = []
    for p in range(2):
        sgp = sg[p, :, :].astype(jnp.float32)
        tgp = tg[p, :, :].astype(jnp.float32)
        ssp = ss[p, :, :].astype(jnp.float32)
        mp = m[:, TFW * p:TFW * (p + 1)].astype(jnp.float32)
        v = (sgp + tgp) * dinv + ssp * icnt + mp + b[:, TFW * p:TFW * (p + 1)]
        cols.append(jnp.maximum(v, 0.0))
    return cols


def _combine_body(degc, cntc, sg, tg, ss, m, b, o):
    cols = _relu_block(degc, cntc, sg, tg, ss, m, b)
    for p in range(2):
        o[:, TFW * p:TFW * (p + 1)] = cols[p]


def _combine(degc, cntc, s_gcn, t_gcn, s_sage, m, bias):
    return pl.pallas_call(
        _combine_body,
        grid=(GRID,),
        in_specs=[
            pl.BlockSpec((BM, 16), lambda i: (i, 0)),
            pl.BlockSpec((BM, 16), lambda i: (i, 0)),
            pl.BlockSpec((2, BM, TFW), lambda i: (0, i, 0)),
            pl.BlockSpec((2, BM, TFW), lambda i: (0, i, 0)),
            pl.BlockSpec((2, BM, TFW), lambda i: (0, i, 0)),
            pl.BlockSpec((BM, 128), lambda i: (i, 0)),
            pl.BlockSpec((1, 128), lambda i: (0, 0)),
        ],
        out_specs=pl.BlockSpec((BM, 128), lambda i: (i, 0)),
        out_shape=jax.ShapeDtypeStruct((N, 128), jnp.float32),
    )(degc, cntc, s_gcn, t_gcn, s_sage, m, bias)


def _combine_final_body(degc, cntc, sg, tg, ss, m, b, w, bo, o):
    cols = _relu_block(degc, cntc, sg, tg, ss, m, b)
    x2 = jnp.concatenate(cols, axis=1)
    o[...] = jnp.dot(x2, w[...], preferred_element_type=jnp.float32) + bo[...]


def _combine_final(degc, cntc, s_gcn, t_gcn, s_sage, m, bias, w, bout):
    return pl.pallas_call(
        _combine_final_body,
        grid=(GRID,),
        in_specs=[
            pl.BlockSpec((BM, 16), lambda i: (i, 0)),
            pl.BlockSpec((BM, 16), lambda i: (i, 0)),
            pl.BlockSpec((2, BM, TFW), lambda i: (0, i, 0)),
            pl.BlockSpec((2, BM, TFW), lambda i: (0, i, 0)),
            pl.BlockSpec((2, BM, TFW), lambda i: (0, i, 0)),
            pl.BlockSpec((BM, 128), lambda i: (i, 0)),
            pl.BlockSpec((1, 128), lambda i: (0, 0)),
            pl.BlockSpec((128, 64), lambda i: (0, 0)),
            pl.BlockSpec((1, 64), lambda i: (0, 0)),
        ],
        out_specs=pl.BlockSpec((BM, 64), lambda i: (i, 0)),
        out_shape=jax.ShapeDtypeStruct((N, 64), jnp.float32),
    )(degc, cntc, s_gcn, t_gcn, s_sage, m, bias, w, bout)


def _final_body(x_ref, w_ref, b_ref, o_ref):
    o_ref[...] = (jnp.dot(x_ref[...], w_ref[...],
                          preferred_element_type=jnp.float32) + b_ref[...])


def _final(x, w, b):
    return pl.pallas_call(
        _final_body,
        grid=(GRID,),
        in_specs=[
            pl.BlockSpec((BM, 128), lambda i: (i, 0)),
            pl.BlockSpec((128, 64), lambda i: (0, 0)),
            pl.BlockSpec((1, 64), lambda i: (0, 0)),
        ],
        out_specs=pl.BlockSpec((BM, 64), lambda i: (i, 0)),
        out_shape=jax.ShapeDtypeStruct((N, 64), jnp.float32),
    )(x, w, b)


def _pad_edges(ei):
    pad = EPAD - E
    src = jnp.concatenate([ei[0].astype(jnp.int32),
                           jnp.zeros((pad,), jnp.int32)])
    dst = jnp.concatenate([ei[1].astype(jnp.int32),
                           jnp.full((pad,), DUMMY, jnp.int32)])
    return src.reshape(NB, BATCH), dst.reshape(NB, BATCH)


def kernel(x_general, x_rainfall, edge_index_gg, edge_index_gr,
           edge_index_rg, edge_index_rr, params):
    s_gg, d_gg = _pad_edges(edge_index_gg)
    s_gr, d_gr = _pad_edges(edge_index_gr)
    s_rg, d_rg = _pad_edges(edge_index_rg)
    s_rr, d_rr = _pad_edges(edge_index_rr)

    counts = _sc_counts(jnp.stack([d_gg, d_rg, d_rr, d_gr]))
    deg_g, cnt_rg = counts[0], counts[1]
    deg_r, cnt_gr = counts[2], counts[3]
    zeros = jnp.zeros((ACC_ROWS, TFW), jnp.bfloat16)

    wg, bg = params['lin_general']
    wr, br = params['lin_rainfall']

    xg, xr = x_general, x_rainfall
    layers = params['layers']
    for li, lp in enumerate(layers):
        tg_g, tl_g, m_g = _mm(deg_g, xg, lp['W_gg'], lp['Wl_gr'], lp['Wr_rg'])
        tg_r, tl_r, m_r = _mm(deg_r, xr, lp['W_rr'], lp['Wl_rg'], lp['Wr_gr'])
        s_gcn_g = _sc_scatter(tg_g, s_gg, d_gg, zeros)
        s_sage_g = _sc_scatter(tl_r, s_rg, d_rg, zeros)
        s_gcn_r = _sc_scatter(tg_r, s_rr, d_rr, zeros)
        s_sage_r = _sc_scatter(tl_g, s_gr, d_gr, zeros)
        bias_g = (lp['b_gg'] + lp['bl_rg']).reshape(1, 128)
        bias_r = (lp['b_rr'] + lp['bl_gr']).reshape(1, 128)
        if li + 1 < len(layers):
            xg = _combine(deg_g, cnt_rg, s_gcn_g, tg_g, s_sage_g, m_g, bias_g)
            xr = _combine(deg_r, cnt_gr, s_gcn_r, tg_r, s_sage_r, m_r, bias_r)
        else:
            gen_out = _combine_final(deg_g, cnt_rg, s_gcn_g, tg_g, s_sage_g,
                                     m_g, bias_g, wg, bg.reshape(1, 64))
            rain_out = _combine_final(deg_r, cnt_gr, s_gcn_r, tg_r, s_sage_r,
                                      m_r, bias_r, wr, br.reshape(1, 64))
    return (gen_out, rain_out)


# speedup vs baseline: 1.2245x; 1.1707x over previous
import functools

import jax
import jax.numpy as jnp
from jax import lax
from jax.experimental import pallas as pl
from jax.experimental.pallas import tpu as pltpu
from jax.experimental.pallas import tpu_sc as plsc

N = 50000
E = 160000
BATCH = 250
NB = 640
NTILES = 16
BT = NB // NTILES
FP = 32
TFW = 64
NPAD = 50048
ACC_ROWS = 50176
ZROWS = 56
ZCOPIES = 56
DRAIN = 3128
ISTAGE = 8
ISTAGE2 = 8
ZTILE = ACC_ROWS // NTILES

BM = 2000
GRID = N // BM

_mesh = plsc.VectorSubcoreMesh(core_axis_name="c", subcore_axis_name="s")


@functools.partial(
    pl.kernel,
    mesh=_mesh,
    out_type=jax.ShapeDtypeStruct((4, NPAD, 16), jnp.float32),
    scratch_types=[
        pltpu.VMEM_SHARED((ACC_ROWS, 16), jnp.float32),
        pltpu.VMEM((ZROWS, 16), jnp.float32),
        pltpu.VMEM((ISTAGE, BATCH), jnp.int32),
        pltpu.VMEM((BATCH, 16), jnp.float32),
    ],
    compiler_params=pltpu.CompilerParams(use_tc_tiling_on_sc=False),
)
def _sc_counts(dst_all, out, acc, zbuf, didx, ones):
    c = lax.axis_index("c")
    s = lax.axis_index("s")
    zero16 = jnp.zeros((16,), jnp.float32)
    one16 = jnp.ones((16,), jnp.float32)

    def fill_z(i, carry):
        zbuf[i, :] = zero16
        return carry
    lax.fori_loop(0, ZROWS, fill_z, 0)

    def fill_o(i, carry):
        ones[i, :] = one16
        return carry
    lax.fori_loop(0, BATCH, fill_o, 0)

    def one_rel(i, carry):
        r = 2 * c + i

        def zcp(k, c2):
            pltpu.sync_copy(zbuf, acc.at[pl.ds(s * (ZROWS * ZCOPIES) + k * ZROWS, ZROWS)])
            return c2
        lax.fori_loop(0, ZCOPIES, zcp, 0)
        plsc.subcore_barrier()

        def chunk(q, c2):
            pltpu.sync_copy(dst_all.at[r].at[pl.ds(s * BT + q * ISTAGE, ISTAGE)], didx)

            def bat(j, c3):
                pltpu.sync_copy(ones, acc.at[didx.at[j]], add=True)
                return c3
            lax.fori_loop(0, ISTAGE, bat, 0)
            return c2
        lax.fori_loop(0, BT // ISTAGE, chunk, 0)
        plsc.subcore_barrier()
        pltpu.sync_copy(acc.at[pl.ds(s * DRAIN, DRAIN)],
                        out.at[r].at[pl.ds(s * DRAIN, DRAIN)])
        plsc.subcore_barrier()
        return carry
    lax.fori_loop(0, 2, one_rel, 0)


@functools.partial(
    pl.kernel,
    mesh=_mesh,
    out_type=jax.ShapeDtypeStruct((2, NPAD, TFW), jnp.bfloat16),
    scratch_types=[
        pltpu.VMEM_SHARED((ACC_ROWS, TFW), jnp.bfloat16),
        pltpu.VMEM((ISTAGE2, BATCH), jnp.int32),
        pltpu.VMEM((ISTAGE2, BATCH), jnp.int32),
        pltpu.VMEM((BATCH, TFW), jnp.bfloat16),
        pltpu.VMEM((BATCH, TFW), jnp.bfloat16),
        pltpu.VMEM((BATCH, TFW), jnp.bfloat16),
        pltpu.SemaphoreType.DMA,
        pltpu.SemaphoreType.DMA,
        pltpu.SemaphoreType.DMA,
        pltpu.SemaphoreType.DMA,
        pltpu.SemaphoreType.DMA,
        pltpu.SemaphoreType.DMA,
    ],
    compiler_params=pltpu.CompilerParams(use_tc_tiling_on_sc=False),
)
def _sc_scatter(tbl, src2d, dst2d, zeros, out,
                acc, sidx, didx, rows0, rows1, rows2,
                gsem0, gsem1, gsem2, ssem0, ssem1, ssem2):
    c = lax.axis_index("c")
    s = lax.axis_index("s")
    p = c
    bufs = (rows0, rows1, rows2)
    gsems = (gsem0, gsem1, gsem2)
    ssems = (ssem0, ssem1, ssem2)

    if True:
        pltpu.sync_copy(zeros.at[pl.ds(s * ZTILE, ZTILE)],
                        acc.at[pl.ds(s * ZTILE, ZTILE)])
        plsc.subcore_barrier()

        def chunk(q, c2):
            pltpu.sync_copy(src2d.at[pl.ds(s * BT + q * ISTAGE2, ISTAGE2)], sidx)
            pltpu.sync_copy(dst2d.at[pl.ds(s * BT + q * ISTAGE2, ISTAGE2)], didx)
            gp = [None] * 3
            sp = [None] * 3
            for j in range(2):
                gp[j] = pltpu.async_copy(tbl.at[p].at[sidx.at[j]], bufs[j], gsems[j])
            for j in range(ISTAGE2):
                b = j % 3
                gp[b].wait()
                sp[b] = pltpu.async_copy(bufs[b], acc.at[didx.at[j]], ssems[b],
                                         add=True)
                if j + 2 < ISTAGE2:
                    bl = (j + 2) % 3
                    if sp[bl] is not None:
                        sp[bl].wait()
                        sp[bl] = None
                    gp[bl] = pltpu.async_copy(tbl.at[p].at[sidx.at[j + 2]],
                                              bufs[bl], gsems[bl])
            for b in range(3):
                if sp[b] is not None:
                    sp[b].wait()
            return c2
        lax.fori_loop(0, BT // ISTAGE2, chunk, 0)
        plsc.subcore_barrier()
        pltpu.sync_copy(acc.at[pl.ds(s * DRAIN, DRAIN)],
                        out.at[p].at[pl.ds(s * DRAIN, DRAIN)])
        plsc.subcore_barrier()


def _mm_body(cnt_ref, x_ref, wg_ref, wl_ref, wr_ref, tg_ref, tl_ref, m_ref):
    x = x_ref[...]
    dinv = lax.rsqrt(cnt_ref[:, :1] + 1.0)
    hg = jnp.dot(x, wg_ref[...], preferred_element_type=jnp.float32) * dinv
    hl = jnp.dot(x, wl_ref[...], preferred_element_type=jnp.float32)
    m_ref[...] = jnp.dot(x, wr_ref[...],
                         preferred_element_type=jnp.float32).astype(jnp.bfloat16)
    for p in range(2):
        tg_ref[p, :, :] = hg[:, TFW * p:TFW * (p + 1)].astype(jnp.bfloat16)
        tl_ref[p, :, :] = hl[:, TFW * p:TFW * (p + 1)].astype(jnp.bfloat16)


def _mm(cnt, x, w_gcn, w_sage_l, w_sage_r):
    return pl.pallas_call(
        _mm_body,
        grid=(GRID,),
        in_specs=[
            pl.BlockSpec((BM, 16), lambda i: (i, 0)),
            pl.BlockSpec((BM, 128), lambda i: (i, 0)),
            pl.BlockSpec((128, 128), lambda i: (0, 0)),
            pl.BlockSpec((128, 128), lambda i: (0, 0)),
            pl.BlockSpec((128, 128), lambda i: (0, 0)),
        ],
        out_specs=[
            pl.BlockSpec((2, BM, TFW), lambda i: (0, i, 0)),
            pl.BlockSpec((2, BM, TFW), lambda i: (0, i, 0)),
            pl.BlockSpec((BM, 128), lambda i: (i, 0)),
        ],
        out_shape=[
            jax.ShapeDtypeStruct((2, N, TFW), jnp.bfloat16),
            jax.ShapeDtypeStruct((2, N, TFW), jnp.bfloat16),
            jax.ShapeDtypeStruct((N, 128), jnp.bfloat16),
        ],
    )(cnt, x, w_gcn, w_sage_l, w_sage_r)


def _relu_block(degc, cntc, sg, tg, ss, m, b):
    dinv = lax.rsqrt(degc[:, :1] + 1.0)
    icnt = 1.0 / jnp.maximum(cntc[:, :1], 1.0)
    cols = []
    for p in range(2):
        sgp = sg[p, :, :].astype(jnp.float32)
        tgp = tg[p, :, :].astype(jnp.float32)
        ssp = ss[p, :, :].astype(jnp.float32)
        mp = m[:, TFW * p:TFW * (p + 1)].astype(jnp.float32)
        v = (sgp + tgp) * dinv + ssp * icnt + mp + b[:, TFW * p:TFW * (p + 1)]
        cols.append(jnp.maximum(v, 0.0))
    return cols


def _combine_body(degc, cntc, sg, tg, ss, m, b, o):
    cols = _relu_block(degc, cntc, sg, tg, ss, m, b)
    for p in range(2):
        o[:, TFW * p:TFW * (p + 1)] = cols[p]


def _combine(degc, cntc, s_gcn, t_gcn, s_sage, m, bias):
    return pl.pallas_call(
        _combine_body,
        grid=(GRID,),
        in_specs=[
            pl.BlockSpec((BM, 16), lambda i: (i, 0)),
            pl.BlockSpec((BM, 16), lambda i: (i, 0)),
            pl.BlockSpec((2, BM, TFW), lambda i: (0, i, 0)),
            pl.BlockSpec((2, BM, TFW), lambda i: (0, i, 0)),
            pl.BlockSpec((2, BM, TFW), lambda i: (0, i, 0)),
            pl.BlockSpec((BM, 128), lambda i: (i, 0)),
            pl.BlockSpec((1, 128), lambda i: (0, 0)),
        ],
        out_specs=pl.BlockSpec((BM, 128), lambda i: (i, 0)),
        out_shape=jax.ShapeDtypeStruct((N, 128), jnp.float32),
    )(degc, cntc, s_gcn, t_gcn, s_sage, m, bias)


def _combine_final_body(degc, cntc, sg, tg, ss, m, b, w, bo, o):
    cols = _relu_block(degc, cntc, sg, tg, ss, m, b)
    x2 = jnp.concatenate(cols, axis=1)
    o[...] = jnp.dot(x2, w[...], preferred_element_type=jnp.float32) + bo[...]


def _combine_final(degc, cntc, s_gcn, t_gcn, s_sage, m, bias, w, bout):
    return pl.pallas_call(
        _combine_final_body,
        grid=(GRID,),
        in_specs=[
            pl.BlockSpec((BM, 16), lambda i: (i, 0)),
            pl.BlockSpec((BM, 16), lambda i: (i, 0)),
            pl.BlockSpec((2, BM, TFW), lambda i: (0, i, 0)),
            pl.BlockSpec((2, BM, TFW), lambda i: (0, i, 0)),
            pl.BlockSpec((2, BM, TFW), lambda i: (0, i, 0)),
            pl.BlockSpec((BM, 128), lambda i: (i, 0)),
            pl.BlockSpec((1, 128), lambda i: (0, 0)),
            pl.BlockSpec((128, 64), lambda i: (0, 0)),
            pl.BlockSpec((1, 64), lambda i: (0, 0)),
        ],
        out_specs=pl.BlockSpec((BM, 64), lambda i: (i, 0)),
        out_shape=jax.ShapeDtypeStruct((N, 64), jnp.float32),
    )(degc, cntc, s_gcn, t_gcn, s_sage, m, bias, w, bout)


def _final_body(x_ref, w_ref, b_ref, o_ref):
    o_ref[...] = (jnp.dot(x_ref[...], w_ref[...],
                          preferred_element_type=jnp.float32) + b_ref[...])


def _final(x, w, b):
    return pl.pallas_call(
        _final_body,
        grid=(GRID,),
        in_specs=[
            pl.BlockSpec((BM, 128), lambda i: (i, 0)),
            pl.BlockSpec((128, 64), lambda i: (0, 0)),
            pl.BlockSpec((1, 64), lambda i: (0, 0)),
        ],
        out_specs=pl.BlockSpec((BM, 64), lambda i: (i, 0)),
        out_shape=jax.ShapeDtypeStruct((N, 64), jnp.float32),
    )(x, w, b)


def _pad_edges(ei):
    return (ei[0].astype(jnp.int32).reshape(NB, BATCH),
            ei[1].astype(jnp.int32).reshape(NB, BATCH))


def kernel(x_general, x_rainfall, edge_index_gg, edge_index_gr,
           edge_index_rg, edge_index_rr, params):
    s_gg, d_gg = _pad_edges(edge_index_gg)
    s_gr, d_gr = _pad_edges(edge_index_gr)
    s_rg, d_rg = _pad_edges(edge_index_rg)
    s_rr, d_rr = _pad_edges(edge_index_rr)

    counts = _sc_counts(jnp.stack([d_gg, d_rg, d_rr, d_gr]))
    deg_g, cnt_rg = counts[0], counts[1]
    deg_r, cnt_gr = counts[2], counts[3]
    zeros = jnp.zeros((ACC_ROWS, TFW), jnp.bfloat16)

    wg, bg = params['lin_general']
    wr, br = params['lin_rainfall']

    xg, xr = x_general, x_rainfall
    layers = params['layers']
    for li, lp in enumerate(layers):
        tg_g, tl_g, m_g = _mm(deg_g, xg, lp['W_gg'], lp['Wl_gr'], lp['Wr_rg'])
        tg_r, tl_r, m_r = _mm(deg_r, xr, lp['W_rr'], lp['Wl_rg'], lp['Wr_gr'])
        s_gcn_g = _sc_scatter(tg_g, s_gg, d_gg, zeros)
        s_sage_g = _sc_scatter(tl_r, s_rg, d_rg, zeros)
        s_gcn_r = _sc_scatter(tg_r, s_rr, d_rr, zeros)
        s_sage_r = _sc_scatter(tl_g, s_gr, d_gr, zeros)
        bias_g = (lp['b_gg'] + lp['bl_rg']).reshape(1, 128)
        bias_r = (lp['b_rr'] + lp['bl_gr']).reshape(1, 128)
        if li + 1 < len(layers):
            xg = _combine(deg_g, cnt_rg, s_gcn_g, tg_g, s_sage_g, m_g, bias_g)
            xr = _combine(deg_r, cnt_gr, s_gcn_r, tg_r, s_sage_r, m_r, bias_r)
        else:
            gen_out = _combine_final(deg_g, cnt_rg, s_gcn_g, tg_g, s_sage_g,
                                     m_g, bias_g, wg, bg.reshape(1, 64))
            rain_out = _combine_final(deg_r, cnt_gr, s_gcn_r, tg_r, s_sage_r,
                                      m_r, bias_r, wr, br.reshape(1, 64))
    return (gen_out, rain_out)
